# baseline (speedup 1.0000x reference)
import functools

import jax
import jax.numpy as jnp
from jax import lax
from jax.experimental import pallas as pl
from jax.experimental.pallas import tpu as pltpu
from jax.experimental.pallas import tpu_sc as plsc

_NUM_GENES = 16906
_EMB_DIM = 200
_N = _NUM_GENES * _EMB_DIM
_NW = 32
_BLOCK = 31_600
_NBLOCKS = _N // _BLOCK
_ITERS = -(-_NBLOCKS // _NW)


def _copy_body(src_hbm, out_hbm, buf):
    wid = lax.axis_index("s") * 2 + lax.axis_index("c")
    for k in range(_ITERS):
        i = wid + k * _NW

        @pl.when(i < _NBLOCKS)
        def _():
            base = pl.multiple_of(i * _BLOCK, 8)
            pltpu.sync_copy(src_hbm.at[pl.ds(base, _BLOCK)], buf)
            pltpu.sync_copy(buf, out_hbm.at[pl.ds(base, _BLOCK)])


def kernel(x, gene_emb):
    del x
    flat = gene_emb.reshape(_N)
    mesh = plsc.VectorSubcoreMesh(core_axis_name="c", subcore_axis_name="s")
    run = functools.partial(
        pl.kernel,
        mesh=mesh,
        out_type=jax.ShapeDtypeStruct((_N,), jnp.float32),
        scratch_types=[pltpu.VMEM((_BLOCK,), jnp.float32)],
    )(_copy_body)
    return run(flat).reshape(_NUM_GENES, _EMB_DIM)


# speedup vs baseline: 3.9386x; 3.9386x over previous
import functools

import jax
import jax.numpy as jnp
from jax import lax
from jax.experimental import pallas as pl
from jax.experimental.pallas import tpu as pltpu
from jax.experimental.pallas import tpu_sc as plsc

_NUM_GENES = 16906
_EMB_DIM = 200
_NW = 32
_ROWS = 264
_ITERS = 2
_TAIL = _NUM_GENES - _NW * _ITERS * _ROWS


def _copy_body(src_hbm, out_hbm, buf, tail_buf):
    wid = lax.axis_index("s") * 2 + lax.axis_index("c")
    for k in range(_ITERS):
        base = pl.multiple_of((wid + k * _NW) * _ROWS, 8)
        pltpu.sync_copy(src_hbm.at[pl.ds(base, _ROWS), :], buf)
        pltpu.sync_copy(buf, out_hbm.at[pl.ds(base, _ROWS), :])

    @pl.when(wid == _NW - 1)
    def _():
        tbase = _NW * _ITERS * _ROWS
        pltpu.sync_copy(src_hbm.at[pl.ds(tbase, _TAIL), :], tail_buf)
        pltpu.sync_copy(tail_buf, out_hbm.at[pl.ds(tbase, _TAIL), :])


def kernel(x, gene_emb):
    del x
    mesh = plsc.VectorSubcoreMesh(core_axis_name="c", subcore_axis_name="s")
    run = functools.partial(
        pl.kernel,
        mesh=mesh,
        out_type=jax.ShapeDtypeStruct((_NUM_GENES, _EMB_DIM), jnp.float32),
        scratch_types=[
            pltpu.VMEM((_ROWS, _EMB_DIM), jnp.float32),
            pltpu.VMEM((_TAIL, _EMB_DIM), jnp.float32),
        ],
    )(_copy_body)
    return run(gene_emb)


# speedup vs baseline: 7.6218x; 1.9352x over previous
import functools

import jax
import jax.numpy as jnp
from jax import lax
from jax.experimental import pallas as pl
from jax.experimental.pallas import tpu as pltpu
from jax.experimental.pallas import tpu_sc as plsc

_NUM_GENES = 16906
_EMB_DIM = 200

_RSLAB = 8
_NSLABS = _EMB_DIM // _RSLAB
_SC0_SLABS = 13


def _copy_body(src_hbm, out_hbm, shared):
    c = lax.axis_index("c")
    s = lax.axis_index("s")
    slab = s + c * _SC0_SLABS
    n_mine = jnp.where(c == 0, _SC0_SLABS, _NSLABS - _SC0_SLABS)

    @pl.when(s < n_mine)
    def _():
        r = pl.multiple_of(slab * _RSLAB, 8)
        pltpu.sync_copy(src_hbm.at[pl.ds(r, _RSLAB), :], shared.at[s])
        pltpu.sync_copy(shared.at[s], out_hbm.at[pl.ds(r, _RSLAB), :])


def kernel(x, gene_emb):
    del x
    src = gene_emb.T
    mesh = plsc.VectorSubcoreMesh(core_axis_name="c", subcore_axis_name="s")
    run = functools.partial(
        pl.kernel,
        mesh=mesh,
        out_type=jax.ShapeDtypeStruct((_EMB_DIM, _NUM_GENES), jnp.float32),
        scratch_types=[
            pltpu.VMEM_SHARED((_SC0_SLABS, _RSLAB, _NUM_GENES), jnp.float32),
        ],
    )(_copy_body)
    return run(src).T
